# Initial kernel scaffold; baseline (speedup 1.0000x reference)
import jax
import jax.numpy as jnp
from jax.experimental import pallas as pl


def kernel(x, edge_index, W1, W2):
    raise NotImplementedError("write your pallas kernel here")


# speedup vs baseline: 5.0362x; 5.0362x over previous
import functools

import jax
import jax.numpy as jnp
from jax import lax
from jax.experimental import pallas as pl
from jax.experimental.pallas import tpu as pltpu
from jax.experimental.pallas import tpu_sc as plsc

N_NODES = 10000
N_EDGES = 320000
NC = 2
NS = 16
NW = NC * NS
EPW = N_EDGES // NW
K = 80
NWIN = EPW // K


def _mm_body(a_ref, w_ref, o_ref):
    o_ref[...] = jnp.dot(a_ref[...], w_ref[...],
                         preferred_element_type=jnp.float32,
                         precision=lax.Precision.HIGHEST)


def _matmul(a, w):
    return pl.pallas_call(
        _mm_body,
        out_shape=jax.ShapeDtypeStruct((a.shape[0], w.shape[1]), jnp.float32),
    )(a, w)


def _mm_combine_body(p_ref, w_ref, o_ref):
    h = p_ref[0] + p_ref[1]
    o_ref[...] = jnp.dot(h, w_ref[...],
                         preferred_element_type=jnp.float32,
                         precision=lax.Precision.HIGHEST)


def _matmul_combine(p, w):
    return pl.pallas_call(
        _mm_combine_body,
        out_shape=jax.ShapeDtypeStruct((p.shape[1], w.shape[1]), jnp.float32),
    )(p, w)


def _add_body(p_ref, o_ref):
    o_ref[...] = p_ref[0] + p_ref[1]


def _add_partials(p):
    return pl.pallas_call(
        _add_body,
        out_shape=jax.ShapeDtypeStruct(p.shape[1:], jnp.float32),
    )(p)


def _sc_aggregate(h, src, dst):
    n, d = h.shape
    rows_per_tile = (n // NS) // 8 * 8 + 8
    mesh = plsc.VectorSubcoreMesh(core_axis_name="c", subcore_axis_name="s")

    @functools.partial(
        pl.kernel,
        out_type=jax.ShapeDtypeStruct((NC, n, d), jnp.float32),
        mesh=mesh,
        scratch_types=[
            pltpu.VMEM_SHARED((n, d), jnp.float32),
            pltpu.VMEM((K,), jnp.int32),
            pltpu.VMEM((K,), jnp.int32),
            pltpu.VMEM((K, d), jnp.float32),
        ],
    )
    def agg(h_hbm, src_hbm, dst_hbm, z_hbm, out_hbm, acc, srcv, dstv, rows):
        c = lax.axis_index("c")
        s = lax.axis_index("s")
        r0 = pl.multiple_of(s * (n // NS) // 8 * 8, 8)
        pltpu.sync_copy(z_hbm.at[pl.ds(r0, rows_per_tile)],
                        acc.at[pl.ds(r0, rows_per_tile)])
        plsc.subcore_barrier()

        base = (c * NS + s) * EPW

        @pl.loop(0, NWIN)
        def _(w):
            off = pl.multiple_of(base + w * K, 8)
            pltpu.sync_copy(src_hbm.at[pl.ds(off, K)], srcv)
            pltpu.sync_copy(dst_hbm.at[pl.ds(off, K)], dstv)
            pltpu.sync_copy(h_hbm.at[srcv], rows)
            pltpu.sync_copy(rows, acc.at[dstv], add=True)

        plsc.subcore_barrier()
        pltpu.sync_copy(acc.at[pl.ds(r0, rows_per_tile)],
                        out_hbm.at[c, pl.ds(r0, rows_per_tile)])

    zeros = jnp.zeros((n, d), jnp.float32)
    return agg(h, src, dst, zeros)


def kernel(x, edge_index, W1, W2):
    ei = edge_index.astype(jnp.int32)
    src = ei[0]
    dst = ei[1]

    h = _matmul(x, W1)
    p1 = _sc_aggregate(h, src, dst)
    a1 = _add_partials(p1)
    p2 = _sc_aggregate(a1, src, dst)
    return _matmul_combine(p2, W2)


# speedup vs baseline: 9.0470x; 1.7964x over previous
import functools

import jax
import jax.numpy as jnp
from jax import lax
from jax.experimental import pallas as pl
from jax.experimental.pallas import tpu as pltpu
from jax.experimental.pallas import tpu_sc as plsc

N_NODES = 10000
N_PAD = N_NODES + 8
N_EDGES = 320000
NC = 2
NS = 16
NW = NC * NS
EPW = N_EDGES // NW
K = 128
NWIN = 80
EPWP = NWIN * K


def _mm_body(a_ref, w_ref, o_ref):
    o_ref[pl.ds(0, N_NODES), :] = jnp.dot(
        a_ref[...], w_ref[...],
        preferred_element_type=jnp.float32, precision=lax.Precision.HIGHEST)
    o_ref[pl.ds(N_NODES, 8), :] = jnp.zeros((8, o_ref.shape[1]), jnp.float32)


def _matmul_pad(a, w):
    return pl.pallas_call(
        _mm_body,
        out_shape=jax.ShapeDtypeStruct((N_PAD, w.shape[1]), jnp.float32),
    )(a, w)


def _mm_combine_body(p_ref, w_ref, o_ref):
    h = p_ref[0] + p_ref[1]
    o_ref[...] = jnp.dot(h, w_ref[...],
                         preferred_element_type=jnp.float32,
                         precision=lax.Precision.HIGHEST)


def _matmul_combine(p, w):
    return pl.pallas_call(
        _mm_combine_body,
        out_shape=jax.ShapeDtypeStruct((p.shape[1], w.shape[1]), jnp.float32),
    )(p, w)


def _add_body(p_ref, o_ref):
    o_ref[pl.ds(0, N_NODES), :] = p_ref[0] + p_ref[1]
    o_ref[pl.ds(N_NODES, 8), :] = jnp.zeros((8, o_ref.shape[1]), jnp.float32)


def _add_partials_pad(p):
    return pl.pallas_call(
        _add_body,
        out_shape=jax.ShapeDtypeStruct((N_PAD, p.shape[2]), jnp.float32),
    )(p)


def _sc_aggregate(h, src2, dst2):
    d = h.shape[1]
    n = N_NODES
    rows_per_tile = (n // NS) // 8 * 8 + 8
    mesh = plsc.VectorSubcoreMesh(core_axis_name="c", subcore_axis_name="s")

    @functools.partial(
        pl.kernel,
        out_type=jax.ShapeDtypeStruct((NC, n, d), jnp.float32),
        mesh=mesh,
        scratch_types=(
            [pltpu.VMEM_SHARED((n, d), jnp.float32)]
            + [pltpu.VMEM((K,), jnp.int32)] * 4
            + [pltpu.VMEM((K, d), jnp.float32)] * 2
            + [pltpu.SemaphoreType.DMA] * 8
        ),
    )
    def agg(h_hbm, src_hbm, dst_hbm, z_hbm, out_hbm, acc,
            sv0, sv1, dv0, dv1, rb0, rb1,
            f0, f1, e0, e1, g0, g1, t0, t1):
        srcv = (sv0, sv1)
        dstv = (dv0, dv1)
        rows = (rb0, rb1)
        fs = (f0, f1)
        es = (e0, e1)
        gs = (g0, g1)
        ss = (t0, t1)
        c = lax.axis_index("c")
        s = lax.axis_index("s")
        wid = c * NS + s

        def src_slice(w):
            return src_hbm.at[wid, pl.ds(pl.multiple_of(w * K, K), K)]

        def dst_slice(w):
            return dst_hbm.at[wid, pl.ds(pl.multiple_of(w * K, K), K)]

        def visit(w, j, first=False, last=False):
            j2 = 1 - j
            pltpu.make_async_copy(h_hbm.at[srcv[j]], rows[j], gs[j]).wait()
            if not last:
                pltpu.async_copy(src_slice(w + 2), srcv[j], fs[j])
            pltpu.make_async_copy(dst_slice(w), dstv[j], es[j]).wait()
            pltpu.async_copy(rows[j], acc.at[dstv[j]], ss[j], add=True)
            if not last:
                pltpu.make_async_copy(src_slice(w + 1), srcv[j2], fs[j2]).wait()
                if not first:
                    pltpu.make_async_copy(rows[j2], acc.at[dstv[j2]],
                                          ss[j2]).wait()
                pltpu.async_copy(dst_slice(w + 1), dstv[j2], es[j2])
                pltpu.async_copy(h_hbm.at[srcv[j2]], rows[j2], gs[j2])

        pltpu.sync_copy(src_slice(0), srcv[0])
        pltpu.async_copy(dst_slice(0), dstv[0], es[0])
        pltpu.async_copy(h_hbm.at[srcv[0]], rows[0], gs[0])
        pltpu.async_copy(src_slice(1), srcv[1], fs[1])
        r0 = pl.multiple_of(s * (n // NS) // 8 * 8, 8)
        pltpu.sync_copy(z_hbm.at[pl.ds(r0, rows_per_tile)],
                        acc.at[pl.ds(r0, rows_per_tile)])
        plsc.subcore_barrier()

        visit(0, 0, first=True)

        @pl.loop(1, NWIN - 3, step=2)
        def _(w):
            visit(w, 1)
            visit(w + 1, 0)

        visit(NWIN - 3, 1)
        visit(NWIN - 2, 0, last=True)
        pltpu.make_async_copy(src_slice(NWIN - 1), srcv[1], fs[1]).wait()
        pltpu.make_async_copy(rows[1], acc.at[dstv[1]], ss[1]).wait()
        pltpu.async_copy(dst_slice(NWIN - 1), dstv[1], es[1])
        pltpu.async_copy(h_hbm.at[srcv[1]], rows[1], gs[1])
        pltpu.make_async_copy(h_hbm.at[srcv[1]], rows[1], gs[1]).wait()
        pltpu.make_async_copy(dst_slice(NWIN - 1), dstv[1], es[1]).wait()
        pltpu.async_copy(rows[1], acc.at[dstv[1]], ss[1], add=True)
        pltpu.make_async_copy(rows[0], acc.at[dstv[0]], ss[0]).wait()
        pltpu.make_async_copy(rows[1], acc.at[dstv[1]], ss[1]).wait()

        plsc.subcore_barrier()
        pltpu.sync_copy(acc.at[pl.ds(r0, rows_per_tile)],
                        out_hbm.at[c, pl.ds(r0, rows_per_tile)])

    zeros = jnp.zeros((n, d), jnp.float32)
    return agg(h, src2, dst2, zeros)


def _pad_edges(ei):
    pad = EPWP - EPW
    lanes = jnp.arange(pad, dtype=jnp.int32)[None, :]
    wids = jnp.arange(NW, dtype=jnp.int32)[:, None]
    src2 = ei[0].reshape(NW, EPW)
    dst2 = ei[1].reshape(NW, EPW)
    src_pad = N_NODES + (lanes + wids) % 8
    dst_pad = (lanes * 131 + wids * 977) % N_NODES
    src2 = jnp.concatenate([src2, jnp.broadcast_to(src_pad, (NW, pad))], 1)
    dst2 = jnp.concatenate([dst2, jnp.broadcast_to(dst_pad, (NW, pad))], 1)
    return src2, dst2


def kernel(x, edge_index, W1, W2):
    ei = edge_index.astype(jnp.int32)
    src2, dst2 = _pad_edges(ei)

    h = _matmul_pad(x, W1)
    p1 = _sc_aggregate(h, src2, dst2)
    a1 = _add_partials_pad(p1)
    p2 = _sc_aggregate(a1, src2, dst2)
    return _matmul_combine(p2, W2)
